# baseline (device time: 50191 ns/iter reference)
import jax
import jax.numpy as jnp
from jax import lax
from jax.experimental import pallas as pl
from jax.experimental.pallas import tpu as pltpu

N_ROWS = 2048
C = 128
PAD = N_ROWS + C
MAX_CHUNKS = PAD // C


def kernel(x, dest):
    n, d = x.shape
    me = lax.axis_index("x")

    is0 = (dest == 0).astype(jnp.int32)
    c0 = jnp.cumsum(is0)
    total0 = c0[-1]
    i = jnp.arange(n, dtype=jnp.int32)
    pos = jnp.where(is0 == 1, c0 - 1, total0 + i - c0)
    perm = jnp.zeros((PAD,), jnp.int32).at[pos].set(i)
    xs = x[perm].astype(jnp.bfloat16)
    kp = jnp.where(me == 0, total0, n - total0)

    def body(kp_ref, x_ref, out_ref, b_ref, send_sems, recv_sems):
        my_x = lax.axis_index("x")
        my_y = lax.axis_index("y")
        partner = (1 - my_x, my_y)

        keep = kp_ref[0]
        k = n - keep

        send_base = jnp.where(my_x == 0, keep, 0)
        s_send = lax.rem(send_base, 8)
        src0 = send_base - s_send
        n_send = (s_send + k + C - 1) // C
        s_recv = jnp.where(my_x == 0, 0, lax.rem(keep, 8))
        n_recv = (s_recv + k + C - 1) // C

        barrier = pltpu.get_barrier_semaphore()
        pl.semaphore_signal(
            barrier, inc=1, device_id=partner,
            device_id_type=pl.DeviceIdType.MESH,
        )
        pl.semaphore_wait(barrier, 1)

        def rdma(j):
            return pltpu.make_async_remote_copy(
                src_ref=x_ref.at[pl.ds(pl.multiple_of(src0 + j * C, 8), C), :],
                dst_ref=b_ref.at[pl.ds(j * C, C), :],
                send_sem=send_sems.at[j],
                recv_sem=recv_sems.at[j],
                device_id=partner,
                device_id_type=pl.DeviceIdType.MESH,
            )

        for j in range(MAX_CHUNKS):
            @pl.when(j < n_send)
            def _():
                rdma(j).start()

        for j in range(MAX_CHUNKS):
            @pl.when(j < n_send)
            def _():
                rdma(j).wait_send()

        for j in range(MAX_CHUNKS):
            @pl.when(j < n_recv)
            def _():
                rdma(j).wait_recv()

        mine = x_ref[: n, :]
        b = b_ref[: n, :]
        shift = jnp.where(my_x == 0, keep - s_recv, lax.rem(n - s_recv, n))
        rolled = pltpu.roll(b, shift, axis=0)
        boundary = jnp.where(my_x == 0, keep, k)
        row = lax.broadcasted_iota(jnp.int32, (n, 1), 0)
        first = jnp.where(my_x == 0, mine, rolled)
        second = jnp.where(my_x == 0, rolled, mine)
        out_ref[...] = jnp.where(row < boundary, first, second)

    return pl.pallas_call(
        body,
        out_shape=jax.ShapeDtypeStruct((n, d), jnp.bfloat16),
        in_specs=[
            pl.BlockSpec(memory_space=pltpu.SMEM),
            pl.BlockSpec(memory_space=pltpu.VMEM),
        ],
        out_specs=pl.BlockSpec(memory_space=pltpu.VMEM),
        scratch_shapes=[
            pltpu.VMEM((PAD, d), jnp.bfloat16),
            pltpu.SemaphoreType.DMA((MAX_CHUNKS,)),
            pltpu.SemaphoreType.DMA((MAX_CHUNKS,)),
        ],
        compiler_params=pltpu.CompilerParams(collective_id=0),
    )(kp.reshape(1), xs)


# device time: 45871 ns/iter; 1.0942x vs baseline; 1.0942x over previous
import jax
import jax.numpy as jnp
from jax import lax
from jax.experimental import pallas as pl
from jax.experimental.pallas import tpu as pltpu

N_ROWS = 2048
C = 128
PAD = N_ROWS + C
MAX_CHUNKS = PAD // C


def kernel(x, dest):
    n, d = x.shape
    me = lax.axis_index("x")

    is0 = (dest == 0).astype(jnp.int32)
    c0 = jnp.cumsum(is0)
    total0 = c0[-1]
    i = jnp.arange(n, dtype=jnp.int32)
    c1 = (i + 1) - c0
    q = i + 1
    ss0 = jnp.searchsorted(c0, q, side="left", method="compare_all")
    ss1 = jnp.searchsorted(c1, q, side="left", method="compare_all")
    perm = jnp.where(i < total0, ss0, jnp.roll(ss1, total0))
    perm = jnp.concatenate([perm, jnp.zeros((PAD - n,), perm.dtype)])
    xs = x[perm].astype(jnp.bfloat16)
    kp = jnp.where(me == 0, total0, n - total0)

    def body(kp_ref, x_ref, out_ref, b_ref, send_sems, recv_sems):
        my_x = lax.axis_index("x")
        my_y = lax.axis_index("y")
        partner = (1 - my_x, my_y)

        keep = kp_ref[0]
        k = n - keep

        send_base = jnp.where(my_x == 0, keep, 0)
        s_send = lax.rem(send_base, 8)
        src0 = send_base - s_send
        n_send = (s_send + k + C - 1) // C
        s_recv = jnp.where(my_x == 0, 0, lax.rem(keep, 8))
        n_recv = (s_recv + k + C - 1) // C

        barrier = pltpu.get_barrier_semaphore()
        pl.semaphore_signal(
            barrier, inc=1, device_id=partner,
            device_id_type=pl.DeviceIdType.MESH,
        )
        pl.semaphore_wait(barrier, 1)

        def rdma(j):
            return pltpu.make_async_remote_copy(
                src_ref=x_ref.at[pl.ds(pl.multiple_of(src0 + j * C, 8), C), :],
                dst_ref=b_ref.at[pl.ds(j * C, C), :],
                send_sem=send_sems.at[j],
                recv_sem=recv_sems.at[j],
                device_id=partner,
                device_id_type=pl.DeviceIdType.MESH,
            )

        for j in range(MAX_CHUNKS):
            @pl.when(j < n_send)
            def _():
                rdma(j).start()

        for j in range(MAX_CHUNKS):
            @pl.when(j < n_send)
            def _():
                rdma(j).wait_send()

        for j in range(MAX_CHUNKS):
            @pl.when(j < n_recv)
            def _():
                rdma(j).wait_recv()

        mine = x_ref[: n, :]
        b = b_ref[: n, :]
        shift = jnp.where(my_x == 0, keep - s_recv, lax.rem(n - s_recv, n))
        rolled = pltpu.roll(b, shift, axis=0)
        boundary = jnp.where(my_x == 0, keep, k)
        row = lax.broadcasted_iota(jnp.int32, (n, 1), 0)
        first = jnp.where(my_x == 0, mine, rolled)
        second = jnp.where(my_x == 0, rolled, mine)
        out_ref[...] = jnp.where(row < boundary, first, second)

    return pl.pallas_call(
        body,
        out_shape=jax.ShapeDtypeStruct((n, d), jnp.bfloat16),
        in_specs=[
            pl.BlockSpec(memory_space=pltpu.SMEM),
            pl.BlockSpec(memory_space=pltpu.VMEM),
        ],
        out_specs=pl.BlockSpec(memory_space=pltpu.VMEM),
        scratch_shapes=[
            pltpu.VMEM((PAD, d), jnp.bfloat16),
            pltpu.SemaphoreType.DMA((MAX_CHUNKS,)),
            pltpu.SemaphoreType.DMA((MAX_CHUNKS,)),
        ],
        compiler_params=pltpu.CompilerParams(collective_id=0),
    )(kp.reshape(1), xs)


# device time: 40659 ns/iter; 1.2344x vs baseline; 1.1282x over previous
import jax
import jax.numpy as jnp
from jax import lax
from jax.experimental import pallas as pl
from jax.experimental.pallas import tpu as pltpu

N_ROWS = 2048
C = 128
PAD = N_ROWS + C
MAX_CHUNKS = 17


def kernel(x, dest):
    n, d = x.shape
    me = lax.axis_index("x")

    is0 = (dest == 0).astype(jnp.int32)
    c0 = jnp.cumsum(is0)
    total0 = c0[-1]
    i = jnp.arange(n, dtype=jnp.int32)
    c1 = (i + 1) - c0
    q = i + 1
    ss0 = jnp.searchsorted(c0, q, side="left", method="compare_all")
    ss1 = jnp.searchsorted(c1, q, side="left", method="compare_all")

    kp = jnp.where(me == 0, total0, n - total0)
    k = n - kp
    s0 = kp % 8
    S0 = jnp.where(s0 > 0, kp - s0 + 8, kp)
    K0 = jnp.where(s0 > 0, k + 8, k)

    zpad = jnp.zeros((PAD - n,), jnp.int32)
    ss0p = jnp.concatenate([ss0, zpad])
    ss1p = jnp.concatenate([ss1, zpad])
    a = jnp.roll(ss0p, jnp.where(me == 0, 0, s0))
    b = jnp.roll(ss1p, jnp.where(me == 0, S0, K0))
    t = jnp.where(me == 0, kp, K0)
    perm = jnp.where(jnp.arange(PAD, dtype=jnp.int32) < t, a, b)
    xs = x[perm].astype(jnp.bfloat16)

    def body(kp_ref, x_ref, out_ref, send_sems, recv_sems, local_sems):
        my_x = lax.axis_index("x")
        my_y = lax.axis_index("y")
        partner = (1 - my_x, my_y)
        im0 = my_x == 0

        kp = kp_ref[0]
        k = n - kp
        s0 = lax.rem(kp, 8)
        s1 = lax.rem(k, 8)
        kp_down = kp - s0
        k_down = k - s1
        k_up = jnp.where(s1 > 0, k_down + 8, k)
        S0 = jnp.where(s0 > 0, kp_down + 8, kp)
        K0 = jnp.where(s0 > 0, k + 8, k)
        D = K0 - k
        F01 = (k_down + C - 1) // C
        F10 = (n - kp_down + C - 1) // C
        L0 = (kp_down + C - 1) // C
        L1 = (n - k_up + C - 1) // C

        def al(v):
            return pl.multiple_of(v, 8)

        def remote(src, dst, j, rows=C):
            return pltpu.make_async_remote_copy(
                src_ref=x_ref.at[pl.ds(al(src), rows), :],
                dst_ref=out_ref.at[pl.ds(al(dst), rows), :],
                send_sem=send_sems.at[j],
                recv_sem=recv_sems.at[j],
                device_id=partner,
                device_id_type=pl.DeviceIdType.MESH,
            )

        def local(src, dst, j):
            return pltpu.make_async_copy(
                x_ref.at[pl.ds(al(src), C), :],
                out_ref.at[pl.ds(al(dst), C), :],
                local_sems.at[j],
            )

        barrier = pltpu.get_barrier_semaphore()
        pl.semaphore_signal(
            barrier, inc=1, device_id=partner,
            device_id_type=pl.DeviceIdType.MESH,
        )
        pl.semaphore_wait(barrier, 1)

        def send01(j):
            src = jnp.where(j == F01 - 1, S0 + k_down - C, S0 + j * C)
            dst = jnp.where(j == F01 - 1, k_down - C, j * C)
            return remote(src, dst, j)

        def send10(j):
            src = jnp.where(j == F10 - 1, n - C - kp_down, j * C)
            dst = jnp.where(j == F10 - 1, n - C, kp_down + j * C)
            return remote(src, dst, j)

        for j in range(MAX_CHUNKS):
            @pl.when(im0 & (j < F01))
            def _():
                send01(j).start()

            @pl.when(im0 & (j == F01) & (s1 > 0))
            def _():
                remote(S0 + k_down, k_down, j, rows=8).start()

            @pl.when((~im0) & (j < F10))
            def _():
                send10(j).start()

        def local0(j):
            off = jnp.where(j == L0 - 1, kp_down - C, j * C)
            return local(off, off, j)

        def local1(j):
            dst = jnp.where(j == L1 - 1, n - C, k_up + j * C)
            return local(dst + D, dst, j)

        for j in range(MAX_CHUNKS):
            @pl.when(im0 & (j < L0))
            def _():
                local0(j).start()

            @pl.when((~im0) & (j < L1))
            def _():
                local1(j).start()

        for j in range(MAX_CHUNKS):
            @pl.when(im0 & (j < L0))
            def _():
                local0(j).wait()

            @pl.when((~im0) & (j < L1))
            def _():
                local1(j).wait()

        for j in range(MAX_CHUNKS):
            @pl.when(im0 & (j < F01))
            def _():
                send01(j).wait_send()

            @pl.when(im0 & (j == F01) & (s1 > 0))
            def _():
                remote(S0 + k_down, k_down, j, rows=8).wait_send()

            @pl.when((~im0) & (j < F10))
            def _():
                send10(j).wait_send()

        for j in range(MAX_CHUNKS):
            @pl.when(im0 & (j < F10))
            def _():
                send10(j).wait_recv()

            @pl.when((~im0) & (j < F01))
            def _():
                send01(j).wait_recv()

            @pl.when((~im0) & (j == F01) & (s1 > 0))
            def _():
                remote(S0 + k_down, k_down, j, rows=8).wait_recv()

        u = lax.broadcasted_iota(jnp.int32, (8, 1), 0)

        @pl.when(im0 & (s0 > 0))
        def _():
            w = out_ref[pl.ds(al(kp_down), 8), :]
            m = x_ref[pl.ds(al(kp_down), 8), :]
            out_ref[pl.ds(al(kp_down), 8), :] = jnp.where(u < s0, m, w)

        @pl.when((~im0) & (s1 > 0))
        def _():
            w = out_ref[pl.ds(al(k_down), 8), :]
            m = x_ref[pl.ds(al(K0 - s1), 8), :]
            out_ref[pl.ds(al(k_down), 8), :] = jnp.where(u < s1, w, m)

    return pl.pallas_call(
        body,
        out_shape=jax.ShapeDtypeStruct((n, d), jnp.bfloat16),
        in_specs=[
            pl.BlockSpec(memory_space=pltpu.SMEM),
            pl.BlockSpec(memory_space=pltpu.VMEM),
        ],
        out_specs=pl.BlockSpec(memory_space=pltpu.VMEM),
        scratch_shapes=[
            pltpu.SemaphoreType.DMA((MAX_CHUNKS,)),
            pltpu.SemaphoreType.DMA((MAX_CHUNKS,)),
            pltpu.SemaphoreType.DMA((MAX_CHUNKS,)),
        ],
        compiler_params=pltpu.CompilerParams(collective_id=0),
    )(kp.reshape(1), xs)
